# baseline (speedup 1.0000x reference)
import functools

import jax
import jax.numpy as jnp
from jax import lax
from jax.experimental import pallas as pl
from jax.experimental.pallas import tpu as pltpu
from jax.experimental.pallas import tpu_sc as plsc

SEQ = 16384
NTAGS = 16
NS = 16
CHUNK = 128
CPT = SEQ // NS // CHUNK

_mesh = plsc.VectorSubcoreMesh(
    core_axis_name="c", subcore_axis_name="s", num_cores=1)


@functools.partial(
    pl.kernel,
    mesh=_mesh,
    compiler_params=pltpu.CompilerParams(use_tc_tiling_on_sc=False),
    out_type=jax.ShapeDtypeStruct((1, NTAGS), jnp.float32),
    scratch_types=[
        pltpu.VMEM((CPT, CHUNK), jnp.int32),
        pltpu.VMEM((CPT, CHUNK, NTAGS), jnp.float32),
        pltpu.VMEM((NS, NTAGS), jnp.float32),
        pltpu.VMEM((1, NTAGS), jnp.float32),
        pltpu.VMEM((NTAGS,), jnp.float32),
        pltpu.VMEM_SHARED((NS, NTAGS), jnp.float32),
        pltpu.SemaphoreType.DMA,
    ],
)
def _bow_sc(x_hbm, table_hbm, bias_hbm, out_hbm,
            idx_v, rows_v, gath_v, stage_v, bias_v, shared, sem):
    sid = lax.axis_index("s")

    pltpu.sync_copy(x_hbm.at[pl.ds(sid * CPT, CPT)], idx_v)

    copies = [
        pltpu.async_copy(table_hbm.at[idx_v.at[j]], rows_v.at[j], sem)
        for j in range(CPT)
    ]
    for c in copies:
        c.wait()

    def body(i, accs):
        return tuple(accs[j] + rows_v[j, i] for j in range(CPT))

    accs = lax.fori_loop(
        0, CHUNK, body,
        tuple(jnp.zeros((NTAGS,), jnp.float32) for _ in range(CPT)))
    acc = accs[0]
    for j in range(1, CPT):
        acc = acc + accs[j]

    stage_v[0] = acc
    pltpu.sync_copy(stage_v, shared.at[pl.ds(sid, 1)])
    plsc.subcore_barrier()

    @pl.when(sid == 0)
    def _():
        pltpu.sync_copy(shared, gath_v)
        pltpu.sync_copy(bias_hbm, bias_v)
        total = bias_v[...]
        for i in range(NS):
            total = total + gath_v[i]
        stage_v[0] = total
        pltpu.sync_copy(stage_v, out_hbm)


def kernel(x, emb_weight, bias):
    x2d = x.reshape(SEQ // CHUNK, CHUNK)
    return _bow_sc(x2d, emb_weight, bias)


# speedup vs baseline: 3.9602x; 3.9602x over previous
import functools

import jax
import jax.numpy as jnp
from jax import lax
from jax.experimental import pallas as pl
from jax.experimental.pallas import tpu as pltpu
from jax.experimental.pallas import tpu_sc as plsc

SEQ = 16384
NTAGS = 16
NWORDS = 1000000
NS = 16
IPT = SEQ // NS
CPT = IPT // 128
NTOT = 1 << 20
SLICE = NTOT // NS

XB = 8192
GRID = (NWORDS + XB - 1) // XB

_mesh = plsc.VectorSubcoreMesh(
    core_axis_name="c", subcore_axis_name="s", num_cores=1)


@functools.partial(
    pl.kernel,
    mesh=_mesh,
    compiler_params=pltpu.CompilerParams(use_tc_tiling_on_sc=False),
    out_type=jax.ShapeDtypeStruct((NTOT,), jnp.float32),
    scratch_types=[
        pltpu.VMEM((CPT, 128), jnp.int32),
        pltpu.VMEM((128,), jnp.float32),
        pltpu.VMEM_SHARED((NTOT,), jnp.float32),
        pltpu.SemaphoreType.DMA,
    ],
)
def _hist_sc(x_hbm, zeros_hbm, out_hbm, idx_v, ones_v, counts_sh, sem):
    sid = lax.axis_index("s")

    pltpu.sync_copy(zeros_hbm.at[pl.ds(sid * SLICE, SLICE)],
                    counts_sh.at[pl.ds(sid * SLICE, SLICE)])
    pltpu.sync_copy(x_hbm.at[pl.ds(sid * CPT, CPT)], idx_v)
    for i in range(8):
        ones_v[pl.ds(i * 16, 16)] = jnp.ones((16,), jnp.float32)
    plsc.subcore_barrier()

    copies = [
        pltpu.async_copy(ones_v, counts_sh.at[idx_v.at[j]], sem, add=True)
        for j in range(CPT)
    ]
    for c in copies:
        c.wait()
    plsc.subcore_barrier()

    pltpu.sync_copy(counts_sh.at[pl.ds(sid * SLICE, SLICE)],
                    out_hbm.at[pl.ds(sid * SLICE, SLICE)])


def _dot_body(t_ref, c_ref, b_ref, o_ref):
    k = pl.program_id(0)
    p = t_ref[...]
    c = c_ref[...]
    pos = lax.broadcasted_iota(jnp.int32, (NTAGS, XB), 1)
    p = jnp.where(pos < NWORDS - k * XB, p, 0.0)
    partial = lax.dot_general(p, c, (((1,), (0,)), ((), ())),
                              preferred_element_type=jnp.float32)

    @pl.when(k == 0)
    def _():
        o_ref[...] = jnp.zeros((NTAGS,), jnp.float32)

    o_ref[...] += partial

    @pl.when(k == GRID - 1)
    def _():
        o_ref[...] += b_ref[...]


_dot_tc = pl.pallas_call(
    _dot_body,
    grid=(GRID,),
    in_specs=[
        pl.BlockSpec((NTAGS, XB), lambda k: (0, k)),
        pl.BlockSpec((XB,), lambda k: (k,)),
        pl.BlockSpec((NTAGS,), lambda k: (0,)),
    ],
    out_specs=pl.BlockSpec((NTAGS,), lambda k: (0,)),
    out_shape=jax.ShapeDtypeStruct((NTAGS,), jnp.float32),
    compiler_params=pltpu.CompilerParams(
        dimension_semantics=("arbitrary",)),
)


def kernel(x, emb_weight, bias):
    table_t = emb_weight.T
    x2d = x.reshape(SEQ // 128, 128)
    zeros = jnp.zeros((NTOT,), jnp.float32)
    counts = _hist_sc(x2d, zeros)
    out = _dot_tc(table_t, counts, bias)
    return out.reshape(1, NTAGS)


# speedup vs baseline: 7.4699x; 1.8862x over previous
import functools

import jax
import jax.numpy as jnp
from jax import lax
from jax.experimental import pallas as pl
from jax.experimental.pallas import tpu as pltpu
from jax.experimental.pallas import tpu_sc as plsc

SEQ = 16384
NTAGS = 16
NWORDS = 1000000
NS = 16
IPT = SEQ // NS
CPT = IPT // 128
NTOT = 1 << 20
SLICE = NTOT // NS

XB = 65536
GRID = (NWORDS + XB - 1) // XB
XJ = XB // 128

_mesh = plsc.VectorSubcoreMesh(
    core_axis_name="c", subcore_axis_name="s", num_cores=1)


@functools.partial(
    pl.kernel,
    mesh=_mesh,
    compiler_params=pltpu.CompilerParams(use_tc_tiling_on_sc=False),
    out_type=jax.ShapeDtypeStruct((NTOT,), jnp.float32),
    scratch_types=[
        pltpu.VMEM((CPT, 128), jnp.int32),
        pltpu.VMEM((128,), jnp.float32),
        pltpu.VMEM_SHARED((NTOT,), jnp.float32),
        pltpu.SemaphoreType.DMA,
    ],
)
def _hist_sc(x_hbm, zeros_hbm, out_hbm, idx_v, ones_v, counts_sh, sem):
    sid = lax.axis_index("s")

    pltpu.sync_copy(zeros_hbm.at[pl.ds(sid * SLICE, SLICE)],
                    counts_sh.at[pl.ds(sid * SLICE, SLICE)])
    pltpu.sync_copy(x_hbm.at[pl.ds(sid * CPT, CPT)], idx_v)
    for i in range(8):
        ones_v[pl.ds(i * 16, 16)] = jnp.ones((16,), jnp.float32)
    plsc.subcore_barrier()

    copies = [
        pltpu.async_copy(ones_v, counts_sh.at[idx_v.at[j]], sem, add=True)
        for j in range(CPT)
    ]
    for c in copies:
        c.wait()
    plsc.subcore_barrier()

    pltpu.sync_copy(counts_sh.at[pl.ds(sid * SLICE, SLICE)],
                    out_hbm.at[pl.ds(sid * SLICE, SLICE)])


def _dot_body(t_ref, c_ref, b_ref, o_ref, acc_ref):
    k = pl.program_id(0)

    @pl.when(k == 0)
    def _():
        acc_ref[...] = jnp.zeros((NTAGS, 128), jnp.float32)

    def accum(p):
        prod = p * c_ref[...][None, :]
        acc_ref[...] += prod.reshape(NTAGS, XJ, 128).sum(axis=1)

    @pl.when(k < GRID - 1)
    def _():
        accum(t_ref[...])

    @pl.when(k == GRID - 1)
    def _():
        pos = lax.broadcasted_iota(jnp.int32, (NTAGS, XB), 1)
        accum(jnp.where(pos < NWORDS - k * XB, t_ref[...], 0.0))
        o_ref[...] = acc_ref[...].sum(axis=1) + b_ref[...]


_dot_tc = pl.pallas_call(
    _dot_body,
    grid=(GRID,),
    in_specs=[
        pl.BlockSpec((NTAGS, XB), lambda k: (0, k)),
        pl.BlockSpec((XB,), lambda k: (k,)),
        pl.BlockSpec((NTAGS,), lambda k: (0,)),
    ],
    out_specs=pl.BlockSpec((NTAGS,), lambda k: (0,)),
    out_shape=jax.ShapeDtypeStruct((NTAGS,), jnp.float32),
    scratch_shapes=[pltpu.VMEM((NTAGS, 128), jnp.float32)],
    compiler_params=pltpu.CompilerParams(
        dimension_semantics=("arbitrary",)),
)


def kernel(x, emb_weight, bias):
    table_t = emb_weight.T
    x2d = x.reshape(SEQ // 128, 128)
    zeros = jnp.zeros((NTOT,), jnp.float32)
    counts = _hist_sc(x2d, zeros)
    out = _dot_tc(table_t, counts, bias)
    return out.reshape(1, NTAGS)


# speedup vs baseline: 7.7517x; 1.0377x over previous
import functools

import jax
import jax.numpy as jnp
from jax import lax
from jax.experimental import pallas as pl
from jax.experimental.pallas import tpu as pltpu
from jax.experimental.pallas import tpu_sc as plsc

SEQ = 16384
NTAGS = 16
NWORDS = 1000000
NS = 16
IPT = SEQ // NS
CPT = IPT // 128
NTOT = 1 << 20
SLICE = NTOT // NS

XB = 131072
GRID = (NWORDS + XB - 1) // XB
XJ = XB // 128

_mesh = plsc.VectorSubcoreMesh(
    core_axis_name="c", subcore_axis_name="s", num_cores=1)


@functools.partial(
    pl.kernel,
    mesh=_mesh,
    compiler_params=pltpu.CompilerParams(use_tc_tiling_on_sc=False),
    out_type=jax.ShapeDtypeStruct((NTOT,), jnp.float32),
    scratch_types=[
        pltpu.VMEM((CPT, 128), jnp.int32),
        pltpu.VMEM((128,), jnp.float32),
        pltpu.VMEM_SHARED((NTOT,), jnp.float32),
        pltpu.SemaphoreType.DMA,
    ],
)
def _hist_sc(x_hbm, zeros_hbm, out_hbm, idx_v, ones_v, counts_sh, sem):
    sid = lax.axis_index("s")

    pltpu.sync_copy(zeros_hbm.at[pl.ds(sid * SLICE, SLICE)],
                    counts_sh.at[pl.ds(sid * SLICE, SLICE)])
    pltpu.sync_copy(x_hbm.at[pl.ds(sid * CPT, CPT)], idx_v)
    for i in range(8):
        ones_v[pl.ds(i * 16, 16)] = jnp.ones((16,), jnp.float32)
    plsc.subcore_barrier()

    copies = [
        pltpu.async_copy(ones_v, counts_sh.at[idx_v.at[j]], sem, add=True)
        for j in range(CPT)
    ]
    for c in copies:
        c.wait()
    plsc.subcore_barrier()

    pltpu.sync_copy(counts_sh.at[pl.ds(sid * SLICE, SLICE)],
                    out_hbm.at[pl.ds(sid * SLICE, SLICE)])


def _dot_body(t_ref, c_ref, b_ref, o_ref, acc_ref):
    k = pl.program_id(0)

    @pl.when(k == 0)
    def _():
        acc_ref[...] = jnp.zeros((NTAGS, 128), jnp.float32)

    def accum(p):
        prod = p * c_ref[...][None, :]
        acc_ref[...] += prod.reshape(NTAGS, XJ, 128).sum(axis=1)

    @pl.when(k < GRID - 1)
    def _():
        accum(t_ref[...])

    @pl.when(k == GRID - 1)
    def _():
        pos = lax.broadcasted_iota(jnp.int32, (NTAGS, XB), 1)
        accum(jnp.where(pos < NWORDS - k * XB, t_ref[...], 0.0))
        o_ref[...] = acc_ref[...].sum(axis=1) + b_ref[...]


_dot_tc = pl.pallas_call(
    _dot_body,
    grid=(GRID,),
    in_specs=[
        pl.BlockSpec((NTAGS, XB), lambda k: (0, k)),
        pl.BlockSpec((XB,), lambda k: (k,)),
        pl.BlockSpec((NTAGS,), lambda k: (0,)),
    ],
    out_specs=pl.BlockSpec((NTAGS,), lambda k: (0,)),
    out_shape=jax.ShapeDtypeStruct((NTAGS,), jnp.float32),
    scratch_shapes=[pltpu.VMEM((NTAGS, 128), jnp.float32)],
    compiler_params=pltpu.CompilerParams(
        dimension_semantics=("arbitrary",)),
)


def kernel(x, emb_weight, bias):
    table_t = emb_weight.T
    x2d = x.reshape(SEQ // 128, 128)
    zeros = jnp.zeros((NTOT,), jnp.float32)
    counts = _hist_sc(x2d, zeros)
    out = _dot_tc(table_t, counts, bias)
    return out.reshape(1, NTAGS)
